# baseline (speedup 1.0000x reference)
import functools

import jax
import jax.numpy as jnp
from jax import lax
from jax.experimental import pallas as pl
from jax.experimental.pallas import tpu as pltpu
from jax.experimental.pallas import tpu_sc as plsc


def _dist_argmin_body(x_ref, w_ref, out_ref, min_sc, idx_sc,
                      *, bm, bk, kblocks):
    j = pl.program_id(0)
    i = pl.program_id(1)
    xb = x_ref[...]
    wb = w_ref[...]
    rows = pl.ds(i * bm, bm)

    cross = jnp.dot(xb * (-2.0), wb, preferred_element_type=jnp.float32,
                    precision=lax.Precision.DEFAULT)
    x_sq = jnp.sum(xb * xb, axis=1, keepdims=True)
    e_sq = jnp.sum(wb * wb, axis=0, keepdims=True)
    d2 = (x_sq + cross) + e_sq
    mn2 = jnp.min(d2, axis=1, keepdims=True)

    s = jnp.sqrt(jnp.maximum(mn2, 0.0))
    c0 = s * s
    c0_bits = lax.bitcast_convert_type(c0, jnp.int32)
    h = jnp.maximum(mn2, c0)
    for off in range(1, 4):
        t_i = lax.bitcast_convert_type(c0_bits + off, jnp.float32)
        ok_i = jnp.sqrt(t_i) == s
        h = jnp.maximum(h, jnp.where(ok_i, t_i, mn2))

    col = lax.broadcasted_iota(jnp.int32, (bm, bk), 1).astype(jnp.float32)
    big = jnp.float32(bk)
    loc = jnp.min(jnp.where(d2 <= h, col, big), axis=1, keepdims=True)
    gidx = loc.astype(jnp.int32) + j * bk

    @pl.when(j == 0)
    def _():
        min_sc[rows, :] = s
        idx_sc[rows, :] = gidx

    @pl.when(j > 0)
    def _():
        run_s = min_sc[rows, :]
        better = s < run_s
        idx_sc[rows, :] = jnp.where(better, gidx, idx_sc[rows, :])
        min_sc[rows, :] = jnp.where(better, s, run_s)

    @pl.when(j == kblocks - 1)
    def _():
        out_ref[...] = jnp.reshape(idx_sc[rows, :], (1, bm // 128, 128))


def _dist_argmin(xt, weight, *, bm=2304, bk=2048):
    m, d = xt.shape
    k = weight.shape[1]
    assert m % bm == 0 and k % bk == 0
    mblocks, kblocks = m // bm, k // bk
    grid = (kblocks, mblocks)
    out = pl.pallas_call(
        functools.partial(_dist_argmin_body, bm=bm, bk=bk, kblocks=kblocks),
        grid=grid,
        in_specs=[
            pl.BlockSpec((bm, d), lambda j, i: (i, 0)),
            pl.BlockSpec((d, bk), lambda j, i: (0, j)),
        ],
        out_specs=pl.BlockSpec((1, bm // 128, 128), lambda j, i: (i, 0, 0)),
        out_shape=jax.ShapeDtypeStruct((m // bm, bm // 128, 128), jnp.int32),
        scratch_shapes=[
            pltpu.VMEM((m, 1), jnp.float32),
            pltpu.VMEM((m, 1), jnp.int32),
        ],
        compiler_params=pltpu.CompilerParams(
            dimension_semantics=("arbitrary", "parallel"),
        ),
    )(xt, weight)
    return out.reshape(m)


_SC_CORES = 2
_SC_SUBCORES = 16
_SC_WORKERS = _SC_CORES * _SC_SUBCORES


def _sc_gather(table, idx):
    v, d = table.shape
    b = idx.shape[0]
    assert b % (8 * _SC_WORKERS) == 0
    bpw = b // _SC_WORKERS
    mesh = plsc.VectorSubcoreMesh(core_axis_name="c", subcore_axis_name="s")

    @functools.partial(
        pl.kernel, mesh=mesh,
        out_type=jax.ShapeDtypeStruct((b, d), jnp.float32),
        scratch_types=[
            pltpu.VMEM((bpw,), jnp.int32),
            pltpu.VMEM((bpw, d), jnp.float32),
            pltpu.SemaphoreType.DMA,
        ],
    )
    def k(table_hbm, idx_hbm, out_hbm, idx_v, rows_v, sem):
        wid = lax.axis_index("s") * _SC_CORES + lax.axis_index("c")
        base = wid * bpw
        pltpu.sync_copy(idx_hbm.at[pl.ds(base, bpw)], idx_v)
        pltpu.async_copy(table_hbm.at[idx_v], rows_v, sem).wait()
        pltpu.sync_copy(rows_v, out_hbm.at[pl.ds(base, bpw)])

    return k(table, idx)


def kernel(x, weight):
    b, d = x.shape[0], x.shape[1]
    spatial = x.shape[2:]
    n = 1
    for s in spatial:
        n *= s
    xt = jnp.transpose(x.reshape(b, d, n), (0, 2, 1)).reshape(b * n, d)
    idx_flat = _dist_argmin(xt, weight)
    gathered = _sc_gather(weight.T, idx_flat)
    shifted = gathered.reshape((b,) + spatial + (d,))
    ndim = x.ndim
    perm = (0, ndim - 1) + tuple(range(1, ndim - 1))
    result = jnp.transpose(shifted, perm)
    argmin = idx_flat.reshape((b,) + spatial)
    return result, argmin


# speedup vs baseline: 1.0687x; 1.0687x over previous
import functools

import jax
import jax.numpy as jnp
from jax import lax
from jax.experimental import pallas as pl
from jax.experimental.pallas import tpu as pltpu
from jax.experimental.pallas import tpu_sc as plsc


def _dist_argmin_body(x_ref, w_ref, out_ref, min_sc, idx_sc,
                      *, bm, bk, kblocks):
    j = pl.program_id(0)
    i = pl.program_id(1)
    xb = x_ref[...]
    wb = w_ref[...]
    rows = pl.ds(i * bm, bm)

    cross = jnp.dot(xb * (-2.0), wb, preferred_element_type=jnp.float32,
                    precision=lax.Precision.DEFAULT)
    x_sq = jnp.sum(xb * xb, axis=1, keepdims=True)
    e_sq = jnp.sum(wb * wb, axis=0, keepdims=True)
    d2 = (x_sq + cross) + e_sq
    mn2 = jnp.min(d2, axis=1, keepdims=True)

    s = jnp.sqrt(jnp.maximum(mn2, 0.0))
    c0 = s * s
    c0_bits = lax.bitcast_convert_type(c0, jnp.int32)
    h = jnp.maximum(mn2, c0)
    for off in range(1, 4):
        t_i = lax.bitcast_convert_type(c0_bits + off, jnp.float32)
        ok_i = jnp.sqrt(t_i) == s
        h = jnp.maximum(h, jnp.where(ok_i, t_i, mn2))

    col = lax.broadcasted_iota(jnp.int32, (bm, bk), 1).astype(jnp.float32)
    big = jnp.float32(bk)
    loc = jnp.min(jnp.where(d2 <= h, col, big), axis=1, keepdims=True)
    gidx = loc.astype(jnp.int32) + j * bk

    @pl.when(j == 0)
    def _():
        min_sc[rows, :] = s
        idx_sc[rows, :] = gidx

    @pl.when(j > 0)
    def _():
        run_s = min_sc[rows, :]
        better = s < run_s
        idx_sc[rows, :] = jnp.where(better, gidx, idx_sc[rows, :])
        min_sc[rows, :] = jnp.where(better, s, run_s)

    @pl.when(j == kblocks - 1)
    def _():
        out_ref[...] = jnp.reshape(idx_sc[rows, :], (1, bm // 128, 128))


def _dist_argmin(xt, weight, *, bm=2304, bk=4096):
    m, d = xt.shape
    k = weight.shape[1]
    assert m % bm == 0 and k % bk == 0
    mblocks, kblocks = m // bm, k // bk
    grid = (kblocks, mblocks)
    out = pl.pallas_call(
        functools.partial(_dist_argmin_body, bm=bm, bk=bk, kblocks=kblocks),
        grid=grid,
        in_specs=[
            pl.BlockSpec((bm, d), lambda j, i: (i, 0)),
            pl.BlockSpec((d, bk), lambda j, i: (0, j)),
        ],
        out_specs=pl.BlockSpec((1, bm // 128, 128), lambda j, i: (i, 0, 0)),
        out_shape=jax.ShapeDtypeStruct((m // bm, bm // 128, 128), jnp.int32),
        scratch_shapes=[
            pltpu.VMEM((m, 1), jnp.float32),
            pltpu.VMEM((m, 1), jnp.int32),
        ],
        compiler_params=pltpu.CompilerParams(
            dimension_semantics=("arbitrary", "parallel"),
        ),
    )(xt, weight)
    return out.reshape(m)


_SC_CORES = 2
_SC_SUBCORES = 16
_SC_WORKERS = _SC_CORES * _SC_SUBCORES


def _sc_gather(table, idx):
    v, d = table.shape
    b = idx.shape[0]
    assert b % (8 * _SC_WORKERS) == 0
    bpw = b // _SC_WORKERS
    mesh = plsc.VectorSubcoreMesh(core_axis_name="c", subcore_axis_name="s")

    @functools.partial(
        pl.kernel, mesh=mesh,
        out_type=jax.ShapeDtypeStruct((b, d), jnp.float32),
        scratch_types=[
            pltpu.VMEM((bpw,), jnp.int32),
            pltpu.VMEM((bpw, d), jnp.float32),
            pltpu.SemaphoreType.DMA,
        ],
    )
    def k(table_hbm, idx_hbm, out_hbm, idx_v, rows_v, sem):
        wid = lax.axis_index("s") * _SC_CORES + lax.axis_index("c")
        base = wid * bpw
        pltpu.sync_copy(idx_hbm.at[pl.ds(base, bpw)], idx_v)
        pltpu.async_copy(table_hbm.at[idx_v], rows_v, sem).wait()
        pltpu.sync_copy(rows_v, out_hbm.at[pl.ds(base, bpw)])

    return k(table, idx)


def kernel(x, weight):
    b, d = x.shape[0], x.shape[1]
    spatial = x.shape[2:]
    n = 1
    for s in spatial:
        n *= s
    xt = jnp.transpose(x.reshape(b, d, n), (0, 2, 1)).reshape(b * n, d)
    idx_flat = _dist_argmin(xt, weight)
    gathered = _sc_gather(weight.T, idx_flat)
    shifted = gathered.reshape((b,) + spatial + (d,))
    ndim = x.ndim
    perm = (0, ndim - 1) + tuple(range(1, ndim - 1))
    result = jnp.transpose(shifted, perm)
    argmin = idx_flat.reshape((b,) + spatial)
    return result, argmin
